# baseline (speedup 1.0000x reference)
import functools

import jax
import jax.numpy as jnp
from jax import lax
from jax.experimental import pallas as pl
from jax.experimental.pallas import tpu as pltpu
from jax.experimental.pallas import tpu_sc as plsc

BATCH = 16384
N_HID = 64
N_TAB = 4
LANES = 16
NUM_CORES = 2
NUM_SUBCORES = 16
NW = NUM_CORES * NUM_SUBCORES
BPW = BATCH // NW
GROUP = 64
GBUF = N_TAB * GROUP
NGRP = BPW // GROUP
WIDE = 2 * N_HID
IDXLEN = BPW + GROUP

_mesh = plsc.VectorSubcoreMesh(core_axis_name="c", subcore_axis_name="s")


@functools.partial(
    pl.kernel,
    mesh=_mesh,
    out_type=jax.ShapeDtypeStruct((BATCH, N_HID), jnp.float32),
    scratch_types=[
        pltpu.VMEM((N_TAB, IDXLEN), jnp.int32),
        pltpu.VMEM((N_TAB, IDXLEN), jnp.int32),
        pltpu.VMEM((GBUF, WIDE), jnp.float32),
        pltpu.VMEM((GBUF, WIDE), jnp.float32),
        pltpu.VMEM((GROUP, N_HID), jnp.float32),
        pltpu.SemaphoreType.DMA,
        pltpu.SemaphoreType.DMA,
    ],
)
def _lookup_sum(tT, e0, e1, e2, e3, out, idx_v, qid_v, gb0, gb1, sbuf,
                sm0, sm1):
    wid = lax.axis_index("s") * NUM_CORES + lax.axis_index("c")
    base = wid * BPW
    tabs = (e0, e1, e2, e3)
    gbs = (gb0, gb1)
    sms = (sm0, sm1)

    for k in range(N_TAB):
        pltpu.sync_copy(tT.at[k, pl.ds(base, BPW)], idx_v.at[k, pl.ds(0, BPW)])
    zeros = jnp.zeros((LANES,), jnp.int32)
    for k in range(N_TAB):
        for h in range(GROUP // LANES):
            idx_v[k, pl.ds(BPW + h * LANES, LANES)] = zeros
    for k in range(N_TAB):
        for h in range(IDXLEN // LANES):
            iv = idx_v[k, pl.ds(h * LANES, LANES)]
            qid_v[k, pl.ds(h * LANES, LANES)] = lax.shift_right_logical(iv, 1)

    def enqueue(g, gbuf, sem):
        row0 = g * GROUP
        for k in range(N_TAB):
            pltpu.async_copy(tabs[k].at[qid_v.at[k, pl.ds(row0, GROUP)]],
                             gbuf.at[pl.ds(k * GROUP, GROUP)], sem)

    def drain(gbuf, sem):
        pltpu.make_async_copy(e0.at[pl.ds(0, GBUF)], gbuf, sem).wait()

    def vsum(g, gbuf):
        row0 = g * GROUP
        for h in range(GROUP // LANES):
            ivs = [idx_v[k, pl.ds(row0 + h * LANES, LANES)]
                   for k in range(N_TAB)]
            for r2 in range(LANES):
                r = h * LANES + r2
                offs = [(ivs[k][r2] & 1) * N_HID for k in range(N_TAB)]
                for j in range(N_HID // LANES):
                    o = j * LANES
                    v = (gbuf[0 * GROUP + r, pl.ds(offs[0] + o, LANES)]
                         + gbuf[1 * GROUP + r, pl.ds(offs[1] + o, LANES)]
                         + gbuf[2 * GROUP + r, pl.ds(offs[2] + o, LANES)]
                         + gbuf[3 * GROUP + r, pl.ds(offs[3] + o, LANES)])
                    sbuf[r, pl.ds(o, LANES)] = v
        pltpu.sync_copy(sbuf, out.at[pl.ds(base + row0, GROUP)])

    enqueue(0, gb0, sm0)

    def pair_body(gg, _):
        b = gg * 2
        for u in range(2):
            g = b + u
            enqueue(g + 1, gbs[(u + 1) % 2], sms[(u + 1) % 2])
            drain(gbs[u], sms[u])
            vsum(g, gbs[u])
        return 0

    lax.fori_loop(0, NGRP // 2, pair_body, 0)
    drain(gbs[NGRP % 2], sms[NGRP % 2])


def kernel(t, emb0, emb1, emb2, emb3):
    tT = t.T.reshape(N_TAB, BATCH)
    wides = [e.reshape(e.shape[0] // 2, WIDE) for e in (emb0, emb1, emb2, emb3)]
    return _lookup_sum(tT, *wides)


# speedup vs baseline: 1.3559x; 1.3559x over previous
import functools

import jax
import jax.numpy as jnp
from jax import lax
from jax.experimental import pallas as pl
from jax.experimental.pallas import tpu as pltpu
from jax.experimental.pallas import tpu_sc as plsc

BATCH = 16384
N_HID = 64
N_TAB = 4
LANES = 16
NUM_CORES = 2
NUM_SUBCORES = 16
NW = NUM_CORES * NUM_SUBCORES
BPW = BATCH // NW
GROUP = 64
GBUF = N_TAB * GROUP
NGRP = BPW // GROUP
IDXLEN = BPW + GROUP

_mesh = plsc.VectorSubcoreMesh(core_axis_name="c", subcore_axis_name="s")


@functools.partial(
    pl.kernel,
    mesh=_mesh,
    out_type=jax.ShapeDtypeStruct((BATCH, N_HID), jnp.float32),
    scratch_types=[
        pltpu.VMEM((N_TAB, IDXLEN), jnp.int32),
        pltpu.VMEM((GBUF, N_HID), jnp.float32),
        pltpu.VMEM((GBUF, N_HID), jnp.float32),
        pltpu.VMEM((GROUP, N_HID), jnp.float32),
        pltpu.SemaphoreType.DMA,
        pltpu.SemaphoreType.DMA,
    ],
)
def _lookup_sum(tT, e0, e1, e2, e3, out, idx_v, rb0, rb1, sbuf, sm0, sm1):
    wid = lax.axis_index("s") * NUM_CORES + lax.axis_index("c")
    base = wid * BPW
    tabs = (e0, e1, e2, e3)
    rbs = (rb0, rb1)
    sms = (sm0, sm1)

    for k in range(N_TAB):
        pltpu.sync_copy(tT.at[k, pl.ds(base, BPW)], idx_v.at[k, pl.ds(0, BPW)])
    zeros = jnp.zeros((LANES,), jnp.int32)
    for k in range(N_TAB):
        for h in range(GROUP // LANES):
            idx_v[k, pl.ds(BPW + h * LANES, LANES)] = zeros

    def enqueue(g, rbuf, sem):
        row0 = g * GROUP
        for h in range(GROUP // LANES):
            iv = [idx_v[k, pl.ds(row0 + h * LANES, LANES)]
                  for k in range(N_TAB)]
            for k in range(N_TAB):
                for r2 in range(LANES):
                    r = h * LANES + r2
                    pltpu.async_copy(tabs[k].at[iv[k][r2]],
                                     rbuf.at[k * GROUP + r], sem)

    def drain(rbuf, sem):
        pltpu.make_async_copy(e0.at[pl.ds(0, GBUF)], rbuf, sem).wait()

    def vsum(g, rbuf):
        row0 = g * GROUP
        for r in range(GROUP):
            for j in range(N_HID // LANES):
                o = j * LANES
                v = (rbuf[0 * GROUP + r, pl.ds(o, LANES)]
                     + rbuf[1 * GROUP + r, pl.ds(o, LANES)]
                     + rbuf[2 * GROUP + r, pl.ds(o, LANES)]
                     + rbuf[3 * GROUP + r, pl.ds(o, LANES)])
                sbuf[r, pl.ds(o, LANES)] = v
        pltpu.sync_copy(sbuf, out.at[pl.ds(base + row0, GROUP)])

    enqueue(0, rb0, sm0)

    def pair_body(gg, _):
        b = gg * 2
        for u in range(2):
            g = b + u
            enqueue(g + 1, rbs[(u + 1) % 2], sms[(u + 1) % 2])
            drain(rbs[u], sms[u])
            vsum(g, rbs[u])
        return 0

    lax.fori_loop(0, NGRP // 2, pair_body, 0)
    drain(rbs[NGRP % 2], sms[NGRP % 2])


def kernel(t, emb0, emb1, emb2, emb3):
    tT = t.T.reshape(N_TAB, BATCH)
    return _lookup_sum(tT, emb0, emb1, emb2, emb3)


# speedup vs baseline: 1.6607x; 1.2248x over previous
import functools

import jax
import jax.numpy as jnp
from jax import lax
from jax.experimental import pallas as pl
from jax.experimental.pallas import tpu as pltpu
from jax.experimental.pallas import tpu_sc as plsc

BATCH = 16384
N_HID = 64
N_TAB = 4
LANES = 16
NUM_CORES = 2
NUM_SUBCORES = 16
NW = NUM_CORES * NUM_SUBCORES
BPW = BATCH // NW
GROUP = 16
GBUF = N_TAB * GROUP
NGRP = BPW // GROUP
IDXLEN = BPW + GROUP

_mesh = plsc.VectorSubcoreMesh(core_axis_name="c", subcore_axis_name="s")


@functools.partial(
    pl.kernel,
    mesh=_mesh,
    out_type=jax.ShapeDtypeStruct((BATCH, N_HID), jnp.float32),
    scratch_types=[
        pltpu.VMEM((N_TAB, IDXLEN), jnp.int32),
        pltpu.VMEM((GBUF, N_HID), jnp.float32),
        pltpu.VMEM((GBUF, N_HID), jnp.float32),
        pltpu.VMEM((BPW, N_HID), jnp.float32),
        pltpu.SemaphoreType.DMA,
        pltpu.SemaphoreType.DMA,
    ],
)
def _lookup_sum(tT, e0, e1, e2, e3, out, idx_v, rb0, rb1, obuf, sm0, sm1):
    wid = lax.axis_index("s") * NUM_CORES + lax.axis_index("c")
    base = wid * BPW
    tabs = (e0, e1, e2, e3)
    rbs = (rb0, rb1)
    sms = (sm0, sm1)

    for k in range(N_TAB):
        pltpu.sync_copy(tT.at[k, pl.ds(base, BPW)], idx_v.at[k, pl.ds(0, BPW)])
    zeros = jnp.zeros((LANES,), jnp.int32)
    for k in range(N_TAB):
        for h in range(GROUP // LANES):
            idx_v[k, pl.ds(BPW + h * LANES, LANES)] = zeros

    def enqueue(g, rbuf, sem):
        row0 = g * GROUP
        for h in range(GROUP // LANES):
            iv = [idx_v[k, pl.ds(row0 + h * LANES, LANES)]
                  for k in range(N_TAB)]
            for k in range(N_TAB):
                for r2 in range(LANES):
                    r = h * LANES + r2
                    pltpu.async_copy(tabs[k].at[iv[k][r2]],
                                     rbuf.at[k * GROUP + r], sem)

    def drain(rbuf, sem):
        pltpu.make_async_copy(e0.at[pl.ds(0, GBUF)], rbuf, sem).wait()

    def vsum(g, rbuf):
        row0 = g * GROUP
        for r in range(GROUP):
            for j in range(N_HID // LANES):
                o = j * LANES
                v = (rbuf[0 * GROUP + r, pl.ds(o, LANES)]
                     + rbuf[1 * GROUP + r, pl.ds(o, LANES)]
                     + rbuf[2 * GROUP + r, pl.ds(o, LANES)]
                     + rbuf[3 * GROUP + r, pl.ds(o, LANES)])
                obuf[row0 + r, pl.ds(o, LANES)] = v

    enqueue(0, rb0, sm0)

    def pair_body(gg, _):
        b = gg * 2
        for u in range(2):
            g = b + u
            enqueue(g + 1, rbs[(u + 1) % 2], sms[(u + 1) % 2])
            drain(rbs[u], sms[u])
            vsum(g, rbs[u])
        return 0

    lax.fori_loop(0, NGRP // 2, pair_body, 0)
    drain(rbs[NGRP % 2], sms[NGRP % 2])
    pltpu.sync_copy(obuf, out.at[pl.ds(base, BPW)])


def kernel(t, emb0, emb1, emb2, emb3):
    tT = t.T.reshape(N_TAB, BATCH)
    return _lookup_sum(tT, emb0, emb1, emb2, emb3)


# speedup vs baseline: 1.7413x; 1.0485x over previous
import functools

import jax
import jax.numpy as jnp
from jax import lax
from jax.experimental import pallas as pl
from jax.experimental.pallas import tpu as pltpu
from jax.experimental.pallas import tpu_sc as plsc

BATCH = 16384
N_HID = 64
N_TAB = 4
LANES = 16
NUM_CORES = 2
NUM_SUBCORES = 16
NW = NUM_CORES * NUM_SUBCORES
BPW = BATCH // NW
GROUP = 8
GBUF = N_TAB * GROUP
NGRP = BPW // GROUP
IDXLEN = BPW + 2 * GROUP

_mesh = plsc.VectorSubcoreMesh(core_axis_name="c", subcore_axis_name="s")


@functools.partial(
    pl.kernel,
    mesh=_mesh,
    out_type=jax.ShapeDtypeStruct((BATCH, N_HID), jnp.float32),
    scratch_types=[
        pltpu.VMEM((N_TAB, IDXLEN), jnp.int32),
        pltpu.VMEM((GBUF, N_HID), jnp.float32),
        pltpu.VMEM((GBUF, N_HID), jnp.float32),
        pltpu.VMEM((BPW, N_HID), jnp.float32),
        pltpu.SemaphoreType.DMA,
        pltpu.SemaphoreType.DMA,
    ],
)
def _lookup_sum(tT, e0, e1, e2, e3, out, idx_v, rb0, rb1, obuf, sm0, sm1):
    wid = lax.axis_index("s") * NUM_CORES + lax.axis_index("c")
    base = wid * BPW
    tabs = (e0, e1, e2, e3)
    rbs = (rb0, rb1)
    sms = (sm0, sm1)

    for k in range(N_TAB):
        pltpu.sync_copy(tT.at[k, pl.ds(base, BPW)], idx_v.at[k, pl.ds(0, BPW)])
    zeros = jnp.zeros((LANES,), jnp.int32)
    for k in range(N_TAB):
        idx_v[k, pl.ds(BPW, LANES)] = zeros

    def enqueue(base16, lo, rbuf, sem):
        iv = [idx_v[k, pl.ds(base16, LANES)] for k in range(N_TAB)]
        for k in range(N_TAB):
            for r in range(GROUP):
                pltpu.async_copy(tabs[k].at[iv[k][lo + r]],
                                 rbuf.at[k * GROUP + r], sem)

    def drain(rbuf, sem):
        pltpu.make_async_copy(e0.at[pl.ds(0, GBUF)], rbuf, sem).wait()

    def vsum(g, rbuf):
        row0 = g * GROUP
        for r in range(GROUP):
            for j in range(N_HID // LANES):
                o = j * LANES
                v = (rbuf[0 * GROUP + r, pl.ds(o, LANES)]
                     + rbuf[1 * GROUP + r, pl.ds(o, LANES)]
                     + rbuf[2 * GROUP + r, pl.ds(o, LANES)]
                     + rbuf[3 * GROUP + r, pl.ds(o, LANES)])
                obuf[row0 + r, pl.ds(o, LANES)] = v

    enqueue(0, 0, rb0, sm0)

    def pair_body(gg, _):
        b = gg * 2
        for u in range(2):
            g = b + u
            enqueue((gg + u) * LANES, GROUP * ((u + 1) % 2),
                    rbs[(u + 1) % 2], sms[(u + 1) % 2])
            drain(rbs[u], sms[u])
            vsum(g, rbs[u])
        return 0

    lax.fori_loop(0, NGRP // 2, pair_body, 0)
    drain(rbs[NGRP % 2], sms[NGRP % 2])
    pltpu.sync_copy(obuf, out.at[pl.ds(base, BPW)])


def kernel(t, emb0, emb1, emb2, emb3):
    tT = t.T.reshape(N_TAB, BATCH)
    return _lookup_sum(tT, emb0, emb1, emb2, emb3)
